# Initial kernel scaffold; baseline (speedup 1.0000x reference)
import jax
import jax.numpy as jnp
from jax.experimental import pallas as pl


def kernel(x, pos, params, edge_index):
    raise NotImplementedError("write your pallas kernel here")


# speedup vs baseline: 7.4330x; 7.4330x over previous
import functools
import math

import jax
import jax.numpy as jnp
import numpy as np
from jax import lax
from jax.experimental import pallas as pl
from jax.experimental.pallas import tpu as pltpu

N1 = 100000
E1 = 1600000
C1 = 16
CHUNK = 2000
GRID1 = N1 // CHUNK
EPS = 1e-5
NEG_INF = float("-inf")


def _cc(gw, gh, sw=240.0, sh=180.0):
    cw, ch = sw / gw, sh / gh
    cx = (np.arange(gw, dtype=np.float32) + 0.5) * cw
    cy = (np.arange(gh, dtype=np.float32) + 0.5) * ch
    xx, yy = np.meshgrid(cx, cy)
    return np.stack([xx.reshape(-1), yy.reshape(-1)], axis=-1).astype(np.float32)


_POSXY1 = _cc(80, 60)
_POSXY2 = _cc(40, 30)
_POSXY3 = _cc(20, 15)


def _t1_body(x_ref, pos_ref, w1h_ref, w1p_ref, wl_ref, bl_ref,
             u1_ref, xsp_ref, st_ref):
    i = pl.program_id(0)
    x = x_ref[...]
    px = pos_ref[:, 0:1]
    py = pos_ref[:, 1:2]
    u1 = x * w1h_ref[...] + px * w1p_ref[0:1, :] + py * w1p_ref[1:2, :]
    u1_ref[...] = u1
    xsp = x * wl_ref[...] + bl_ref[...]
    xsp_ref[...] = xsp

    @pl.when(i == 0)
    def _():
        st_ref[...] = jnp.zeros_like(st_ref)

    st_ref[0:1, :] += jnp.sum(xsp, axis=0, keepdims=True)
    st_ref[1:2, :] += jnp.sum(xsp * xsp, axis=0, keepdims=True)


def _t1(x, pos, w1h, w1p, wl, bl):
    return pl.pallas_call(
        _t1_body,
        grid=(GRID1,),
        in_specs=[
            pl.BlockSpec((CHUNK, 1), lambda i: (i, 0)),
            pl.BlockSpec((CHUNK, 3), lambda i: (i, 0)),
            pl.BlockSpec((1, C1), lambda i: (0, 0)),
            pl.BlockSpec((2, C1), lambda i: (0, 0)),
            pl.BlockSpec((1, C1), lambda i: (0, 0)),
            pl.BlockSpec((1, C1), lambda i: (0, 0)),
        ],
        out_specs=[
            pl.BlockSpec((CHUNK, C1), lambda i: (i, 0)),
            pl.BlockSpec((CHUNK, C1), lambda i: (i, 0)),
            pl.BlockSpec((8, C1), lambda i: (0, 0)),
        ],
        out_shape=[
            jax.ShapeDtypeStruct((N1, C1), jnp.float32),
            jax.ShapeDtypeStruct((N1, C1), jnp.float32),
            jax.ShapeDtypeStruct((8, C1), jnp.float32),
        ],
    )(x, pos, w1h, w1p, wl, bl)


def _t2_body(mh_ref, pos_ref, wp_ref, b_ref, agg_ref, st_ref):
    i = pl.program_id(0)
    m = jnp.maximum(mh_ref[0], mh_ref[1])
    px = pos_ref[:, 0:1]
    py = pos_ref[:, 1:2]
    pc = px * wp_ref[0:1, :] + py * wp_ref[1:2, :]
    t = m - pc + b_ref[...]
    agg = jnp.where(jnp.isfinite(t), t, 0.0)
    agg_ref[...] = agg

    @pl.when(i == 0)
    def _():
        st_ref[...] = jnp.zeros_like(st_ref)

    st_ref[0:1, :] += jnp.sum(agg, axis=0, keepdims=True)
    st_ref[1:2, :] += jnp.sum(agg * agg, axis=0, keepdims=True)


def _t2(mh, pos, wp, b):
    return pl.pallas_call(
        _t2_body,
        grid=(GRID1,),
        in_specs=[
            pl.BlockSpec((2, CHUNK, C1), lambda i: (0, i, 0)),
            pl.BlockSpec((CHUNK, 3), lambda i: (i, 0)),
            pl.BlockSpec((2, C1), lambda i: (0, 0)),
            pl.BlockSpec((1, C1), lambda i: (0, 0)),
        ],
        out_specs=[
            pl.BlockSpec((CHUNK, C1), lambda i: (i, 0)),
            pl.BlockSpec((8, C1), lambda i: (0, 0)),
        ],
        out_shape=[
            jax.ShapeDtypeStruct((N1, C1), jnp.float32),
            jax.ShapeDtypeStruct((8, C1), jnp.float32),
        ],
    )(mh, pos, wp, b)


def _t3_body(agg_ref, st_ref, pos_ref, w2h_ref, w2p_ref, g_ref, be_ref,
             u2_ref):
    mean = st_ref[0:1, :] / N1
    var = st_ref[1:2, :] / N1 - mean * mean
    h = (agg_ref[...] - mean) * lax.rsqrt(var + EPS) * g_ref[...] + be_ref[...]
    h = jnp.maximum(h, 0.0)
    px = pos_ref[:, 0:1]
    py = pos_ref[:, 1:2]
    u2 = jnp.dot(h, w2h_ref[...], preferred_element_type=jnp.float32)
    u2_ref[...] = u2 + px * w2p_ref[0:1, :] + py * w2p_ref[1:2, :]


def _t3(agg, st, pos, w2h, w2p, g, be):
    return pl.pallas_call(
        _t3_body,
        grid=(GRID1,),
        in_specs=[
            pl.BlockSpec((CHUNK, C1), lambda i: (i, 0)),
            pl.BlockSpec((8, C1), lambda i: (0, 0)),
            pl.BlockSpec((CHUNK, 3), lambda i: (i, 0)),
            pl.BlockSpec((C1, C1), lambda i: (0, 0)),
            pl.BlockSpec((2, C1), lambda i: (0, 0)),
            pl.BlockSpec((1, C1), lambda i: (0, 0)),
            pl.BlockSpec((1, C1), lambda i: (0, 0)),
        ],
        out_specs=pl.BlockSpec((CHUNK, C1), lambda i: (i, 0)),
        out_shape=jax.ShapeDtypeStruct((N1, C1), jnp.float32),
    )(agg, st, pos, w2h, w2p, g, be)


def _t5_body(agg_ref, st2_ref, xsp_ref, stx_ref, g2_ref, be2_ref,
             gl_ref, bel_ref, pos_ref, h_ref, c_ref):
    m2 = st2_ref[0:1, :] / N1
    v2 = st2_ref[1:2, :] / N1 - m2 * m2
    h2 = (agg_ref[...] - m2) * lax.rsqrt(v2 + EPS) * g2_ref[...] + be2_ref[...]
    mx = stx_ref[0:1, :] / N1
    vx = stx_ref[1:2, :] / N1 - mx * mx
    xs = (xsp_ref[...] - mx) * lax.rsqrt(vx + EPS) * gl_ref[...] + bel_ref[...]
    h_ref[...] = jnp.maximum(h2 + xs, 0.0)
    px = pos_ref[:, 0:1]
    py = pos_ref[:, 1:2]
    gx = jnp.clip(jnp.floor(px / 3.0), 0, 79).astype(jnp.int32)
    gy = jnp.clip(jnp.floor(py / 3.0), 0, 59).astype(jnp.int32)
    cell = gy * 80 + gx
    c_ref[...] = jnp.broadcast_to(cell.T, (8, cell.shape[0]))[None]


def _t5(agg2, st2, xsp, stx, g2, be2, gl, bel, pos):
    return pl.pallas_call(
        _t5_body,
        grid=(GRID1,),
        in_specs=[
            pl.BlockSpec((CHUNK, C1), lambda i: (i, 0)),
            pl.BlockSpec((8, C1), lambda i: (0, 0)),
            pl.BlockSpec((CHUNK, C1), lambda i: (i, 0)),
            pl.BlockSpec((8, C1), lambda i: (0, 0)),
            pl.BlockSpec((1, C1), lambda i: (0, 0)),
            pl.BlockSpec((1, C1), lambda i: (0, 0)),
            pl.BlockSpec((1, C1), lambda i: (0, 0)),
            pl.BlockSpec((1, C1), lambda i: (0, 0)),
            pl.BlockSpec((CHUNK, 3), lambda i: (i, 0)),
        ],
        out_specs=[pl.BlockSpec((CHUNK, C1), lambda i: (i, 0)),
                   pl.BlockSpec((1, 8, CHUNK), lambda i: (i, 0, 0))],
        out_shape=[jax.ShapeDtypeStruct((N1, C1), jnp.float32),
                   jax.ShapeDtypeStruct((GRID1, 8, CHUNK), jnp.int32)],
    )(agg2, st2, xsp, stx, g2, be2, gl, bel, pos)


def _bn_full(x, g, b):
    m = jnp.mean(x, axis=0, keepdims=True)
    v = jnp.mean(x * x, axis=0, keepdims=True) - m * m
    return (x - m) * lax.rsqrt(v + EPS) * g + b


def _stencil_max(u, gh, gw):
    c = u.shape[-1]
    u3 = u.reshape(gh, gw, c)
    p = jnp.pad(u3, ((1, 1), (1, 1), (0, 0)), constant_values=NEG_INF)
    m = jnp.full_like(u3, NEG_INF)
    for dy in (-1, 0, 1):
        for dx in (-1, 0, 1):
            if dx == 0 and dy == 0:
                continue
            m = jnp.maximum(m, p[1 - dy:1 - dy + gh, 1 - dx:1 - dx + gw, :])
    return m.reshape(gh * gw, c)


NW = 14


def _pack_block(p, cin):
    return (p['Wl'].T, p['bl'][None, :], p['gl'][None, :], p['bel'][None, :],
            p['W1'][:, :cin].T, p['W1'][:, cin:].T, p['b1'][None, :],
            p['g1'][None, :], p['be1'][None, :],
            p['W2'][:, :p['W2'].shape[0]].T, p['W2'][:, p['W2'].shape[0]:].T,
            p['b2'][None, :], p['g2'][None, :], p['be2'][None, :])


def _grid_block(x, pxy, w, gh, gw):
    px, py = pxy[:, 0:1], pxy[:, 1:2]
    pc1 = px * w[5][0:1, :] + py * w[5][1:2, :]
    pc2 = px * w[10][0:1, :] + py * w[10][1:2, :]
    xs = _bn_full(jnp.dot(x, w[0], preferred_element_type=jnp.float32)
                  + w[1], w[2], w[3])
    u = jnp.dot(x, w[4], preferred_element_type=jnp.float32) + pc1
    m = _stencil_max(u, gh, gw)
    h = _bn_full(m - pc1 + w[6], w[7], w[8])
    h = jnp.maximum(h, 0.0)
    u = jnp.dot(h, w[9], preferred_element_type=jnp.float32) + pc2
    m = _stencil_max(u, gh, gw)
    h = _bn_full(m - pc2 + w[11], w[12], w[13])
    return jnp.maximum(h + xs, 0.0)


def _pool4(h, t, gh, gw):
    c = h.shape[-1]
    ht = jnp.concatenate([h, t], axis=-1)
    r = ht.reshape(gh // 2, 2, gw // 2, 2, c + 1)
    s = (r[:, 0, :, 0, :] + r[:, 0, :, 1, :]
         + r[:, 1, :, 0, :] + r[:, 1, :, 1, :]) * 0.25
    s = s.reshape((gh // 2) * (gw // 2), c + 1)
    hp = s[:, :c]
    out = hp * (hp > 0.0).astype(hp.dtype)
    return out, s[:, c:c + 1]


def _t6_body(*refs):
    acch_ref, acca_ref = refs[0:2]
    pxy1_ref, pxy2_ref, pxy3_ref = refs[2:5]
    w2 = tuple(r[...] for r in refs[5:5 + NW])
    w3 = tuple(r[...] for r in refs[5 + NW:5 + 2 * NW])
    w4 = tuple(r[...] for r in refs[5 + 2 * NW:5 + 3 * NW])
    s0_ref, s1_ref, s2_ref = refs[5 + 3 * NW:]

    s = acch_ref[0:4800, :]
    aux = acca_ref[0:4800, :]
    cnt = jnp.maximum(aux[:, 1:2], 1.0)
    v = s / cnt
    h0 = v * (v > 0.0).astype(v.dtype)
    t1 = aux[:, 0:1] / cnt

    s0 = _grid_block(h0, pxy1_ref[...], w2, 60, 80)
    s0_ref[...] = s0
    h, t2 = _pool4(s0, t1, 60, 80)
    s1 = _grid_block(h, pxy2_ref[...], w3, 30, 40)
    s1_ref[...] = s1
    h, _ = _pool4(s1, t2, 30, 40)
    s2_ref[...] = _grid_block(h, pxy3_ref[...], w4, 15, 20)


def _t6(acch, acca, params):
    def full(shape):
        return pl.BlockSpec(shape, lambda: tuple(0 for _ in shape))

    w2 = _pack_block(params['b2'], 16)
    w3 = _pack_block(params['b3'], 32)
    w4 = _pack_block(params['b4'], 64)
    flat_w = list(w2) + list(w3) + list(w4)
    return pl.pallas_call(
        _t6_body,
        in_specs=[
            full((4864, 16)), full((4864, 16)),
            full((4800, 2)), full((1200, 2)), full((300, 2)),
        ] + [full(a.shape) for a in flat_w],
        out_specs=[full((4800, 32)), full((1200, 64)), full((300, 128))],
        out_shape=[
            jax.ShapeDtypeStruct((4800, 32), jnp.float32),
            jax.ShapeDtypeStruct((1200, 64), jnp.float32),
            jax.ShapeDtypeStruct((300, 128), jnp.float32),
        ],
    )(acch, acca, jnp.asarray(_POSXY1), jnp.asarray(_POSXY2),
      jnp.asarray(_POSXY3), *flat_w)


from jax.experimental.pallas import tpu_sc as plsc

NPAD = 100352
WNODE = 3136
PCHUNK = 448
NCP = 4864
DUMP = 4800

NB = 16
ROWS_B = 6256
NSEG = NB * ROWS_B
EW = E1 // 32
ESUB = 2000
FLUSH = 512
BCAP = 50176


def _pool1_sc(h_pad, t_pad, cells_pad):
    mesh = plsc.VectorSubcoreMesh(core_axis_name="c", subcore_axis_name="s")

    @functools.partial(
        pl.kernel, mesh=mesh,
        compiler_params=pltpu.CompilerParams(needs_layout_passes=False),
        out_type=[jax.ShapeDtypeStruct((32 * NCP * 16,), jnp.float32),
                  jax.ShapeDtypeStruct((32 * NCP * 16,), jnp.float32)],
        scratch_types=[
            pltpu.VMEM((PCHUNK * 16,), jnp.float32),
            pltpu.VMEM((NCP * 16,), jnp.float32),
            pltpu.VMEM((PCHUNK,), jnp.int32),
            pltpu.VMEM((PCHUNK,), jnp.float32),
        ],
    )
    def k(h_hbm, t_hbm, c_hbm, outH, outA, hrows, acc, cv, tv):
        cid = lax.axis_index("c")
        sid = lax.axis_index("s")
        wid = sid * 2 + cid
        iota = lax.iota(jnp.int32, 16)
        zrow = jnp.zeros((16,), jnp.float32)
        lane0 = (iota == 0).astype(jnp.float32)
        lane1 = (iota == 1).astype(jnp.float32)

        def zero_acc(i, _):
            acc[pl.ds(i * 16, 16)] = zrow
            return 0

        def load_chunk(kk, with_h):
            base = wid * WNODE + kk * PCHUNK
            if with_h:
                pltpu.sync_copy(h_hbm.at[pl.ds(base * 16, PCHUNK * 16)],
                                hrows)
            pltpu.sync_copy(c_hbm.at[pl.ds(base, PCHUNK)], cv)
            pltpu.sync_copy(t_hbm.at[pl.ds(base, PCHUNK)], tv)
            return base

        lax.fori_loop(0, NCP, zero_acc, 0)

        def chunkA(kk, _):
            base = load_chunk(kk, True)

            def vgroup(v, _):
                o = v * 16
                cvv = cv[pl.ds(o, 16)]
                for j in range(16):
                    o2 = cvv[j] * 16
                    acc[pl.ds(o2, 16)] = acc[pl.ds(o2, 16)] \
                        + hrows[pl.ds((o + j) * 16, 16)]
                return 0
            lax.fori_loop(0, PCHUNK // 16, vgroup, 0)
            return 0
        lax.fori_loop(0, WNODE // PCHUNK, chunkA, 0)
        pltpu.sync_copy(acc, outH.at[pl.ds(wid * NCP * 16, NCP * 16)])

        lax.fori_loop(0, NCP, zero_acc, 0)

        def chunkB(kk, _):
            base = load_chunk(kk, False)

            def vgroup(v, _):
                o = v * 16
                cvv = cv[pl.ds(o, 16)]
                tvv = tv[pl.ds(o, 16)]
                for j in range(16):
                    o2 = cvv[j] * 16
                    rv = lane0 * tvv[j] + lane1
                    acc[pl.ds(o2, 16)] = acc[pl.ds(o2, 16)] + rv
                return 0
            lax.fori_loop(0, PCHUNK // 16, vgroup, 0)
            return 0
        lax.fori_loop(0, WNODE // PCHUNK, chunkB, 0)
        pltpu.sync_copy(acc, outA.at[pl.ds(wid * NCP * 16, NCP * 16)])

    aH, aA = k(h_pad.reshape(-1), t_pad, cells_pad)
    return _tred(aH.reshape(32 * NCP, 16)), _tred(aA.reshape(32 * NCP, 16))


def _tred_body(x_ref, o_ref):
    i = pl.program_id(0)

    @pl.when(i == 0)
    def _():
        o_ref[...] = jnp.zeros_like(o_ref)

    o_ref[...] += x_ref[...]


def _tred(x):
    return pl.pallas_call(
        _tred_body,
        grid=(32,),
        in_specs=[pl.BlockSpec((NCP, 16), lambda i: (i, 0))],
        out_specs=pl.BlockSpec((NCP, 16), lambda i: (0, 0)),
        out_shape=jax.ShapeDtypeStruct((NCP, 16), jnp.float32),
    )(x)


def _tedge_body(src_ref, dst_ref, q_ref, pk_ref):
    d = dst_ref[...]
    sc = src_ref[...]
    q = (d.astype(jnp.float32) * (1.0 / ROWS_B)).astype(jnp.int32)
    r = d - q * ROWS_B
    q = q + (r >= ROWS_B).astype(jnp.int32) - (r < 0).astype(jnp.int32)
    r = d - q * ROWS_B
    q_ref[...] = q
    pk_ref[...] = sc | (r << 17)


def _tedge(edge_index):
    src2d = edge_index[0].reshape(800, 2000)
    dst2d = edge_index[1].reshape(800, 2000)
    q2d, pk2d = pl.pallas_call(
        _tedge_body,
        grid=(25,),
        in_specs=[pl.BlockSpec((32, 2000), lambda i: (i, 0)),
                  pl.BlockSpec((32, 2000), lambda i: (i, 0))],
        out_specs=[pl.BlockSpec((32, 2000), lambda i: (i, 0)),
                   pl.BlockSpec((32, 2000), lambda i: (i, 0))],
        out_shape=[jax.ShapeDtypeStruct((800, 2000), jnp.int32),
                   jax.ShapeDtypeStruct((800, 2000), jnp.int32)],
    )(src2d, dst2d)
    return q2d.reshape(-1), pk2d.reshape(-1)


ESUB2 = 400


def _segmax_sc(u, q_flat, pk_flat):
    mesh = plsc.VectorSubcoreMesh(core_axis_name="c", subcore_axis_name="s")

    @functools.partial(
        pl.kernel, mesh=mesh,
        compiler_params=pltpu.CompilerParams(needs_layout_passes=False,
                                             use_tc_tiling_on_sc=False),
        out_type=[jax.ShapeDtypeStruct((2 * NSEG * 16,), jnp.float32),
                  jax.ShapeDtypeStruct((32 * NB * BCAP,), jnp.int32),
                  jax.ShapeDtypeStruct((512,), jnp.int32)],
        scratch_types=[
            pltpu.VMEM((NB * 528,), jnp.int32),
            pltpu.VMEM((16,), jnp.int32),
            pltpu.VMEM(((ROWS_B + 1) * 16,), jnp.float32),
            pltpu.VMEM((FLUSH,), jnp.int32),
            pltpu.VMEM((FLUSH,), jnp.int32),
            pltpu.VMEM((FLUSH, 16), jnp.float32),
            pltpu.VMEM((ESUB2,), jnp.int32),
            pltpu.VMEM((ESUB2,), jnp.int32),
            pltpu.SMEM((NB,), jnp.int32),
            pltpu.SMEM((NB,), jnp.int32),
            pltpu.SemaphoreType.DMA,
        ],
    )
    def k(u_hbm, q_hbm, pk_hbm, out, regions, counts_hbm, sbuf, cvec, acc,
          pkb, sidx, rows, qv, pv, cnt_s, fl_s, sem):
        cid = lax.axis_index("c")
        sid = lax.axis_index("s")
        wid = sid * 2 + cid
        iotac = lax.iota(jnp.int32, 16)
        lane0 = iotac == 0
        lmask = [iotac == j for j in range(16)]

        for b in range(NB):
            cnt_s[b] = 0
            fl_s[b] = 0

        def subchunk(sc, _):
            base = wid * EW + sc * ESUB2
            pltpu.sync_copy(q_hbm.at[pl.ds(base, ESUB2)], qv)
            pltpu.sync_copy(pk_hbm.at[pl.ds(base, ESUB2)], pv)

            def vgroup(v, _):
                o = v * 16
                qvec = qv[pl.ds(o, 16)]
                pvec = pv[pl.ds(o, 16)]
                for j in range(16):
                    qj = qvec[j]
                    cnt = cnt_s[qj]
                    plsc.store_scatter(
                        sbuf, [jnp.full((16,), qj * 528 + cnt, jnp.int32)],
                        pvec, mask=lmask[j])
                    cnt = cnt + 1
                    cnt_s[qj] = cnt

                    @pl.when(cnt >= FLUSH)
                    def _():
                        fli = fl_s[qj]
                        roff = (wid * NB + qj) * BCAP + fli * FLUSH
                        pltpu.sync_copy(
                            sbuf.at[pl.ds(qj * 528, FLUSH)],
                            regions.at[pl.ds(roff, FLUSH)])
                        sbuf[pl.ds(qj * 528, 16)] = \
                            sbuf[pl.ds(qj * 528 + FLUSH, 16)]
                        fl_s[qj] = fli + 1
                        cnt_s[qj] = cnt - FLUSH
                return 0
            lax.fori_loop(0, ESUB2 // 16, vgroup, 0)
            return 0
        lax.fori_loop(0, EW // ESUB2, subchunk, 0)

        for b in range(NB):
            fli = fl_s[b]
            roff = (wid * NB + b) * BCAP + fli * FLUSH
            pltpu.sync_copy(sbuf.at[pl.ds(b * 528, FLUSH)],
                            regions.at[pl.ds(roff, FLUSH)])
            tot = fli * FLUSH + cnt_s[b]
            plsc.store_scatter(cvec, [jnp.full((16,), b, jnp.int32)],
                               jnp.full((16,), tot, jnp.int32), mask=lane0)
        pltpu.sync_copy(cvec, counts_hbm.at[pl.ds(wid * 16, 16)])
        plsc.subcore_barrier()

        ninf = jnp.full((16,), NEG_INF, jnp.float32)

        def initrow(i, _):
            acc[pl.ds(i * 16, 16)] = ninf
            return 0
        lax.fori_loop(0, ROWS_B + 1, initrow, 0)

        nmax = jnp.full((16,), N1 - 1, jnp.int32)
        for ts in range(16):
            w2 = ts * 2 + cid
            pltpu.sync_copy(counts_hbm.at[pl.ds(w2 * 16, 16)], cvec)
            crow = cvec[...]
            cnt = crow[0]
            for j in range(1, 16):
                cnt = jnp.where(sid == j, crow[j], cnt)

            def chunk(j, _):
                off = j * FLUSH
                roff = (w2 * NB + sid) * BCAP + off
                pltpu.sync_copy(regions.at[pl.ds(roff, FLUSH)], pkb)

                def unp(v, _):
                    o = v * 16
                    p16 = pkb[pl.ds(o, 16)]
                    sidx[pl.ds(o, 16)] = jnp.minimum(p16 & 0x1FFFF, nmax)
                    return 0
                lax.fori_loop(0, FLUSH // 16, unp, 0)
                pltpu.async_copy(u_hbm.at[sidx], rows, sem).wait()
                m = jnp.minimum(cnt - off, FLUSH)

                def vgroup(v, _):
                    o = v * 16
                    wv = pkb[pl.ds(o, 16)]
                    dlv = jnp.minimum((wv >> 17) & 0x1FFF, ROWS_B - 1)
                    dlv = jnp.where(o + iotac < m, dlv, ROWS_B)
                    for j in range(16):
                        o2 = dlv[j] * 16
                        rv = plsc.load_gather(
                            rows, [jnp.full((16,), o + j, jnp.int32), iotac])
                        acc[pl.ds(o2, 16)] = jnp.maximum(
                            acc[pl.ds(o2, 16)], rv)
                    return 0
                lax.fori_loop(0, FLUSH // 16, vgroup, 0)
                return 0
            lax.fori_loop(0, (cnt + FLUSH - 1) // FLUSH, chunk, 0)

        pltpu.sync_copy(
            acc.at[pl.ds(0, ROWS_B * 16)],
            out.at[pl.ds((cid * NSEG + sid * ROWS_B) * 16, ROWS_B * 16)])

    mh = k(u, q_flat, pk_flat)[0]
    return mh.reshape(2, NSEG, 16)[:, :N1, :]


def _segmax_xla(u, src, dst):
    m = jax.ops.segment_max(u[src], dst, num_segments=N1)
    mh = jnp.stack([m, jnp.full_like(m, NEG_INF)])
    return mh


def _pool1_xla(h, pos):
    gx = jnp.clip(jnp.floor(pos[:, 0] / 3.0), 0, 79).astype(jnp.int32)
    gy = jnp.clip(jnp.floor(pos[:, 1] / 3.0), 0, 59).astype(jnp.int32)
    cell = gy * 80 + gx
    sh = jax.ops.segment_sum(h, cell, num_segments=4800)
    aux = jax.ops.segment_sum(
        jnp.concatenate([pos[:, 2:3], jnp.ones((N1, 1), jnp.float32),
                         jnp.zeros((N1, 14), jnp.float32)], axis=1),
        cell, num_segments=4800)
    pad = jnp.zeros((8, 16), jnp.float32)
    acch = jnp.stack([jnp.concatenate([sh, pad], 0),
                      jnp.zeros((4808, 16), jnp.float32)])
    acca = jnp.stack([jnp.concatenate([aux, pad], 0),
                      jnp.zeros((4808, 16), jnp.float32)])
    return acch, acca


def kernel(x, pos, params, edge_index):
    p1 = params['b1']
    w1h = p1['W1'][:, :1].T
    w1p = p1['W1'][:, 1:3].T
    wl = p1['Wl'].T
    bl = p1['bl'][None, :]
    b1 = p1['b1'][None, :]
    b2 = p1['b2'][None, :]
    w2h = p1['W2'][:, :C1].T
    w2p = p1['W2'][:, C1:].T

    u1, xsp, stx = _t1(x, pos, w1h, w1p, wl, bl)
    qf, pf = _tedge(edge_index)
    mh1 = _segmax_sc(u1, qf, pf)
    agg1, st1 = _t2(mh1, pos, w1p, b1)
    u2 = _t3(agg1, st1, pos, w2h, w2p, p1['g1'][None, :], p1['be1'][None, :])
    mh2 = _segmax_sc(u2, qf, pf)
    agg2, st2 = _t2(mh2, pos, w2p, b2)
    h, cells8 = _t5(agg2, st2, xsp, stx, p1['g2'][None, :],
                    p1['be2'][None, :], p1['gl'][None, :],
                    p1['bel'][None, :], pos)
    h_pad = jnp.concatenate([h, jnp.zeros((NPAD - N1, C1), jnp.float32)], 0)
    t_pad = jnp.concatenate([pos[:, 2], jnp.zeros((NPAD - N1,), jnp.float32)])
    cells_pad = jnp.concatenate(
        [cells8[:, 0, :].reshape(N1), jnp.full((NPAD - N1,), DUMP,
                                               jnp.int32)])
    acch, acca = _pool1_sc(h_pad, t_pad, cells_pad)
    s0, s1, s2 = _t6(acch, acca, params)
    return (s0, s1, s2)
